# baseline (device time: 191679 ns/iter reference)
import jax
import jax.numpy as jnp
from jax import lax
from jax.experimental import pallas as pl
from jax.experimental.pallas import tpu as pltpu

M = 16384
N = 2048
HALF = N // 2
HROWS = M // 2
NC = 16
CM = HROWS // NC
NSEND = 8
NZS = 8
QMAX = 6.0
SCALE = 127.0 / QMAX
INV_SCALE = QMAX / 127.0


def kernel(x):
    xf = x[0]

    def body(x_ref, out_ref, recv_ref,
             y_send_sems, y_recv_sems, z_send_sems, z_recv_sems,
             vload, vsend, va, vb, vout, csems):
        mx = lax.axis_index("x")
        my = lax.axis_index("y")
        mz = lax.axis_index("z")
        peer = (mx, 1 - my, mz)
        znbr = (1 - mx, my, mz)

        peer_col = (1 - my) * HALF
        my_col = my * HALF
        r_direct = mx * HROWS
        r_fwd = (1 - mx) * HROWS

        def y_rdma(c):
            r0 = r_direct + c * CM
            return pltpu.make_async_remote_copy(
                src_ref=vsend.at[c % NSEND],
                dst_ref=recv_ref.at[pl.ds(r0, CM), :],
                send_sem=y_send_sems.at[c % NSEND],
                recv_sem=y_recv_sems.at[c],
                device_id=peer,
                device_id_type=pl.DeviceIdType.MESH,
            )

        def z_fwd_rdma(c):
            r0 = r_direct + c * CM
            return pltpu.make_async_remote_copy(
                src_ref=recv_ref.at[pl.ds(r0, CM), :],
                dst_ref=recv_ref.at[pl.ds(r0, CM), :],
                send_sem=z_send_sems.at[c % NZS],
                recv_sem=z_recv_sems.at[c],
                device_id=znbr,
                device_id_type=pl.DeviceIdType.MESH,
            )

        def z_in_rdma(c):
            r0 = r_fwd + c * CM
            return pltpu.make_async_remote_copy(
                src_ref=recv_ref.at[pl.ds(r0, CM), :],
                dst_ref=recv_ref.at[pl.ds(r0, CM), :],
                send_sem=z_send_sems.at[0],
                recv_sem=z_recv_sems.at[c],
                device_id=znbr,
                device_id_type=pl.DeviceIdType.MESH,
            )

        def load_chunk(c):
            r0 = r_direct + c * CM
            cp = pltpu.make_async_copy(
                x_ref.at[pl.ds(r0, CM), pl.ds(peer_col, HALF)],
                vload.at[c % 2], csems.at[c % 2])
            cp.start()
            return cp

        ld = load_chunk(0)
        barrier_sem = pltpu.get_barrier_semaphore()
        for nbr in (peer, znbr):
            pl.semaphore_signal(barrier_sem, inc=1, device_id=nbr,
                                device_id_type=pl.DeviceIdType.MESH)
        pl.semaphore_wait(barrier_sem, 2)
        for c in range(NC):
            ld.wait()
            nxt = load_chunk(c + 1) if c + 1 < NC else None
            if c >= NSEND:
                y_rdma(c - NSEND).wait_send()
            vsend[c % NSEND] = jnp.rint(
                jnp.clip(vload[c % 2] * SCALE, -127.0, 127.0)
            ).astype(jnp.int8)
            y_rdma(c).start()
            ld = nxt

        seq = [("y", 0), ("y", 1)]
        for c in range(NC - 2):
            seq += [("f", c), ("y", c + 2)]
        seq += [("f", NC - 2), ("f", NC - 1)]

        def item_r0(it):
            kind, c = it
            return (r_direct if kind == "y" else r_fwd) + c * CM

        def ready(it):
            kind, c = it
            if kind == "y":
                y_rdma(c).wait_recv()
                if c >= NZS:
                    z_fwd_rdma(c - NZS).wait_send()
                z_fwd_rdma(c).start()
            else:
                z_in_rdma(c).wait_recv()

        def issue_loads(it, s):
            r0 = item_r0(it)
            cp_a = pltpu.make_async_copy(
                x_ref.at[pl.ds(r0, CM), pl.ds(my_col, HALF)],
                va.at[s], csems.at[2 + s])
            cp_b = pltpu.make_async_copy(
                recv_ref.at[pl.ds(r0, CM), :], vb.at[s], csems.at[4 + s])
            cp_a.start()
            cp_b.start()
            return cp_a, cp_b

        n_items = len(seq)
        ready(seq[0])
        loads = issue_loads(seq[0], 0)
        stores = [None, None]
        for k in range(n_items):
            s = k % 2
            if k + 1 < n_items:
                ready(seq[k + 1])
                next_loads = issue_loads(seq[k + 1], (k + 1) % 2)
            cp_a, cp_b = loads
            cp_a.wait()
            cp_b.wait()
            if stores[s] is not None:
                stores[s].wait()
            vout[s] = va[s] + vb[s].astype(jnp.float32) * INV_SCALE
            cp_o = pltpu.make_async_copy(
                vout.at[s], out_ref.at[pl.ds(item_r0(seq[k]), CM), :],
                csems.at[6 + s])
            cp_o.start()
            stores[s] = cp_o
            if k + 1 < n_items:
                loads = next_loads
        stores[0].wait()
        stores[1].wait()

        for c in range(NC - NSEND, NC):
            y_rdma(c).wait_send()
        for c in range(NC - NZS, NC):
            z_fwd_rdma(c).wait_send()

    out, _ = pl.pallas_call(
        body,
        out_shape=[
            jax.ShapeDtypeStruct((M, HALF), jnp.float32),
            jax.ShapeDtypeStruct((M, HALF), jnp.int8),
        ],
        in_specs=[pl.BlockSpec(memory_space=pl.ANY)],
        out_specs=[
            pl.BlockSpec(memory_space=pl.ANY),
            pl.BlockSpec(memory_space=pl.ANY),
        ],
        scratch_shapes=[
            pltpu.SemaphoreType.DMA((NSEND,)),
            pltpu.SemaphoreType.DMA((NC,)),
            pltpu.SemaphoreType.DMA((NZS,)),
            pltpu.SemaphoreType.DMA((NC,)),
            pltpu.VMEM((2, CM, HALF), jnp.float32),
            pltpu.VMEM((NSEND, CM, HALF), jnp.int8),
            pltpu.VMEM((2, CM, HALF), jnp.float32),
            pltpu.VMEM((2, CM, HALF), jnp.int8),
            pltpu.VMEM((2, CM, HALF), jnp.float32),
            pltpu.SemaphoreType.DMA((8,)),
        ],
        compiler_params=pltpu.CompilerParams(
            collective_id=0, vmem_limit_bytes=60 * 1024 * 1024),
    )(xf)
    return out


# device time: 164234 ns/iter; 1.1671x vs baseline; 1.1671x over previous
import jax
import jax.numpy as jnp
from jax import lax
from jax.experimental import pallas as pl
from jax.experimental.pallas import tpu as pltpu

M = 16384
N = 2048
HALF = N // 2
HROWS = M // 2
NC = 8
CM = HROWS // NC
NSEND = 8
NZS = 8
QMAX = 6.0
SCALE = 127.0 / QMAX
INV_SCALE = QMAX / 127.0


def kernel(x):
    xf = x[0]

    def body(x_ref, out_ref, recv_ref,
             y_send_sems, y_recv_sems, z_send_sems, z_recv_sems,
             vload, vsend, va, vb, vout, csems):
        mx = lax.axis_index("x")
        my = lax.axis_index("y")
        mz = lax.axis_index("z")
        peer = (mx, 1 - my, mz)
        znbr = (1 - mx, my, mz)

        peer_col = (1 - my) * HALF
        my_col = my * HALF
        r_direct = mx * HROWS
        r_fwd = (1 - mx) * HROWS

        def y_rdma(c):
            r0 = r_direct + c * CM
            return pltpu.make_async_remote_copy(
                src_ref=vsend.at[c % NSEND],
                dst_ref=recv_ref.at[pl.ds(r0, CM), :],
                send_sem=y_send_sems.at[c % NSEND],
                recv_sem=y_recv_sems.at[c],
                device_id=peer,
                device_id_type=pl.DeviceIdType.MESH,
            )

        def z_fwd_rdma(c):
            r0 = r_direct + c * CM
            return pltpu.make_async_remote_copy(
                src_ref=recv_ref.at[pl.ds(r0, CM), :],
                dst_ref=recv_ref.at[pl.ds(r0, CM), :],
                send_sem=z_send_sems.at[c % NZS],
                recv_sem=z_recv_sems.at[c],
                device_id=znbr,
                device_id_type=pl.DeviceIdType.MESH,
            )

        def z_in_rdma(c):
            r0 = r_fwd + c * CM
            return pltpu.make_async_remote_copy(
                src_ref=recv_ref.at[pl.ds(r0, CM), :],
                dst_ref=recv_ref.at[pl.ds(r0, CM), :],
                send_sem=z_send_sems.at[0],
                recv_sem=z_recv_sems.at[c],
                device_id=znbr,
                device_id_type=pl.DeviceIdType.MESH,
            )

        def load_chunk(c):
            r0 = r_direct + c * CM
            cp = pltpu.make_async_copy(
                x_ref.at[pl.ds(r0, CM), pl.ds(peer_col, HALF)],
                vload.at[c % 2], csems.at[c % 2])
            cp.start()
            return cp

        ld = load_chunk(0)
        barrier_sem = pltpu.get_barrier_semaphore()
        for nbr in (peer, znbr):
            pl.semaphore_signal(barrier_sem, inc=1, device_id=nbr,
                                device_id_type=pl.DeviceIdType.MESH)
        pl.semaphore_wait(barrier_sem, 2)
        for c in range(NC):
            ld.wait()
            nxt = load_chunk(c + 1) if c + 1 < NC else None
            if c >= NSEND:
                y_rdma(c - NSEND).wait_send()
            vsend[c % NSEND] = jnp.rint(
                jnp.clip(vload[c % 2] * SCALE, -127.0, 127.0)
            ).astype(jnp.int8)
            y_rdma(c).start()
            ld = nxt

        seq = [("y", 0), ("y", 1)]
        for c in range(NC - 2):
            seq += [("f", c), ("y", c + 2)]
        seq += [("f", NC - 2), ("f", NC - 1)]

        def item_r0(it):
            kind, c = it
            return (r_direct if kind == "y" else r_fwd) + c * CM

        def ready(it):
            kind, c = it
            if kind == "y":
                y_rdma(c).wait_recv()
                if c >= NZS:
                    z_fwd_rdma(c - NZS).wait_send()
                z_fwd_rdma(c).start()
            else:
                z_in_rdma(c).wait_recv()

        def issue_loads(it, s):
            r0 = item_r0(it)
            cp_a = pltpu.make_async_copy(
                x_ref.at[pl.ds(r0, CM), pl.ds(my_col, HALF)],
                va.at[s], csems.at[2 + s])
            cp_b = pltpu.make_async_copy(
                recv_ref.at[pl.ds(r0, CM), :], vb.at[s], csems.at[4 + s])
            cp_a.start()
            cp_b.start()
            return cp_a, cp_b

        n_items = len(seq)
        ready(seq[0])
        loads = issue_loads(seq[0], 0)
        stores = [None, None]
        for k in range(n_items):
            s = k % 2
            if k + 1 < n_items:
                ready(seq[k + 1])
                next_loads = issue_loads(seq[k + 1], (k + 1) % 2)
            cp_a, cp_b = loads
            cp_a.wait()
            cp_b.wait()
            if stores[s] is not None:
                stores[s].wait()
            vout[s] = va[s] + vb[s].astype(jnp.float32) * INV_SCALE
            cp_o = pltpu.make_async_copy(
                vout.at[s], out_ref.at[pl.ds(item_r0(seq[k]), CM), :],
                csems.at[6 + s])
            cp_o.start()
            stores[s] = cp_o
            if k + 1 < n_items:
                loads = next_loads
        stores[0].wait()
        stores[1].wait()

        for c in range(NC - NSEND, NC):
            y_rdma(c).wait_send()
        for c in range(NC - NZS, NC):
            z_fwd_rdma(c).wait_send()

    out, _ = pl.pallas_call(
        body,
        out_shape=[
            jax.ShapeDtypeStruct((M, HALF), jnp.float32),
            jax.ShapeDtypeStruct((M, HALF), jnp.int8),
        ],
        in_specs=[pl.BlockSpec(memory_space=pl.ANY)],
        out_specs=[
            pl.BlockSpec(memory_space=pl.ANY),
            pl.BlockSpec(memory_space=pl.ANY),
        ],
        scratch_shapes=[
            pltpu.SemaphoreType.DMA((NSEND,)),
            pltpu.SemaphoreType.DMA((NC,)),
            pltpu.SemaphoreType.DMA((NZS,)),
            pltpu.SemaphoreType.DMA((NC,)),
            pltpu.VMEM((2, CM, HALF), jnp.float32),
            pltpu.VMEM((NSEND, CM, HALF), jnp.int8),
            pltpu.VMEM((2, CM, HALF), jnp.float32),
            pltpu.VMEM((2, CM, HALF), jnp.int8),
            pltpu.VMEM((2, CM, HALF), jnp.float32),
            pltpu.SemaphoreType.DMA((8,)),
        ],
        compiler_params=pltpu.CompilerParams(
            collective_id=0, vmem_limit_bytes=60 * 1024 * 1024),
    )(xf)
    return out
